# Initial kernel scaffold; baseline (speedup 1.0000x reference)
import jax
import jax.numpy as jnp
from jax.experimental import pallas as pl


def kernel(input, table):
    raise NotImplementedError("write your pallas kernel here")


# speedup vs baseline: 2.9702x; 2.9702x over previous
import functools

import jax
import jax.numpy as jnp
from jax import lax
from jax.experimental import pallas as pl
from jax.experimental.pallas import tpu as pltpu
from jax.experimental.pallas import tpu_sc as plsc

NC = 2
NS = 16
NW = NC * NS
L = 16

BATCH = 16384
HIST = 50
EMBED_DIM = 32
HPAD = 64

BLKI = 2 * HIST
NBLK = 16
STEP_I = NBLK * BLKI
NSTEP = BATCH * HIST // (NW * STEP_I)


def _body(idx_hbm, table_hbm, hidx_hbm, out_hbm,
          idx_v, rows_v, acc, hidx_v, shared, sem0, sem1):
  c = lax.axis_index("c")
  s = lax.axis_index("s")
  w = s * NC + c
  sems = (sem0, sem1)

  zero = jnp.zeros((L,), jnp.float32)

  def zbody(h, carry):
    acc[h, pl.ds(0, L)] = zero
    acc[h, pl.ds(L, L)] = zero
    return carry
  lax.fori_loop(0, HPAD, zbody, 0)

  @pl.when(s == 0)
  def _():
    pltpu.sync_copy(acc, shared)

  pltpu.sync_copy(hidx_hbm, hidx_v)

  def fire(buf, g):
    pltpu.sync_copy(idx_hbm.at[w * NSTEP + g], idx_v.at[buf])
    for j in range(NBLK):
      pltpu.async_copy(table_hbm.at[idx_v.at[buf, j]], rows_v.at[buf, j],
                       sems[buf])

  def drain(buf):
    for j in range(NBLK):
      pltpu.make_async_copy(table_hbm.at[idx_v.at[buf, j]],
                            rows_v.at[buf, j], sems[buf]).wait()

  def accum(buf):
    r = rows_v.at[buf]

    def hbody(h, carry):
      lo = acc[h, pl.ds(0, L)]
      hi = acc[h, pl.ds(L, L)]
      for j in range(NBLK):
        lo = lo + r[j, h, pl.ds(0, L)]
        hi = hi + r[j, h, pl.ds(L, L)]
        lo = lo + r[j, h + HIST, pl.ds(0, L)]
        hi = hi + r[j, h + HIST, pl.ds(L, L)]
      acc[h, pl.ds(0, L)] = lo
      acc[h, pl.ds(L, L)] = hi
      return carry
    lax.fori_loop(0, HIST, hbody, 0)

  fire(0, 0)

  def gbody(i, carry):
    g = i * 2
    fire(1, g + 1)
    drain(0)
    accum(0)

    @pl.when(g + 2 < NSTEP)
    def _():
      fire(0, g + 2)
    drain(1)
    accum(1)
    return carry
  lax.fori_loop(0, NSTEP // 2, gbody, 0)

  plsc.subcore_barrier()
  pltpu.sync_copy(acc, shared.at[hidx_v], add=True)
  plsc.subcore_barrier()

  @pl.when(s == 0)
  def _():
    pltpu.sync_copy(shared, acc)
    scale = jnp.full((L,), 1.0 / BATCH, jnp.float32)

    def sbody(h, carry):
      acc[h, pl.ds(0, L)] = acc[h, pl.ds(0, L)] * scale
      acc[h, pl.ds(L, L)] = acc[h, pl.ds(L, L)] * scale
      return carry
    lax.fori_loop(0, HIST, sbody, 0)
    pltpu.sync_copy(acc.at[pl.ds(0, HIST)], out_hbm.at[c])


_sc_call = functools.partial(
    pl.kernel,
    out_type=jax.ShapeDtypeStruct((NC, HIST, EMBED_DIM), jnp.float32),
    mesh=plsc.VectorSubcoreMesh(core_axis_name="c", subcore_axis_name="s"),
    compiler_params=pltpu.CompilerParams(use_tc_tiling_on_sc=False),
    scratch_types=[
        pltpu.VMEM((2, NBLK, BLKI), jnp.int32),
        pltpu.VMEM((2, NBLK, BLKI, EMBED_DIM), jnp.float32),
        pltpu.VMEM((HPAD, EMBED_DIM), jnp.float32),
        pltpu.VMEM((HPAD,), jnp.int32),
        pltpu.VMEM_SHARED((HPAD, EMBED_DIM), jnp.float32),
        pltpu.SemaphoreType.DMA,
        pltpu.SemaphoreType.DMA,
    ],
)(_body)


@jax.jit
def kernel(input, table):
  idx = input.reshape(NW * NSTEP, NBLK, BLKI)
  hidx = jnp.arange(HPAD, dtype=jnp.int32)
  partial = _sc_call(idx, table, hidx)
  return partial.sum(axis=0)


# speedup vs baseline: 2.9705x; 1.0001x over previous
import functools

import jax
import jax.numpy as jnp
from jax import lax
from jax.experimental import pallas as pl
from jax.experimental.pallas import tpu as pltpu
from jax.experimental.pallas import tpu_sc as plsc

NC = 2
NS = 16
NW = NC * NS
L = 16

BATCH = 16384
HIST = 50
VOCAB = 1000000
EMBED_DIM = 32
HPAD = 64

BLKI = 2 * HIST
NBLK = 16
STEP_I = NBLK * BLKI
NSTEP = BATCH * HIST // (NW * STEP_I)


def _body(idx_hbm, table_hbm, hidx_hbm, out_hbm,
          idx_v, rows_v, acc, hidx_v, shared, sem0, sem1):
  c = lax.axis_index("c")
  s = lax.axis_index("s")
  w = s * NC + c
  sems = (sem0, sem1)

  zero = jnp.zeros((L,), jnp.float32)

  def zbody(h, carry):
    acc[h, pl.ds(0, L)] = zero
    acc[h, pl.ds(L, L)] = zero
    return carry
  lax.fori_loop(0, HPAD, zbody, 0)

  @pl.when(s == 0)
  def _():
    pltpu.sync_copy(acc, shared)

  pltpu.sync_copy(hidx_hbm, hidx_v)

  def fire(buf, g):
    pltpu.sync_copy(idx_hbm.at[w * NSTEP + g], idx_v.at[buf])
    for j in range(NBLK):
      pltpu.async_copy(table_hbm.at[idx_v.at[buf, j]], rows_v.at[buf, j],
                       sems[buf])

  def drain(buf):
    for j in range(NBLK):
      pltpu.make_async_copy(table_hbm.at[idx_v.at[buf, j]],
                            rows_v.at[buf, j], sems[buf]).wait()

  def accum(buf):
    r = rows_v.at[buf]

    def hbody(h, carry):
      lo = acc[h, pl.ds(0, L)]
      hi = acc[h, pl.ds(L, L)]
      for j in range(NBLK):
        lo = lo + r[j, h, pl.ds(0, L)]
        hi = hi + r[j, h, pl.ds(L, L)]
        lo = lo + r[j, h + HIST, pl.ds(0, L)]
        hi = hi + r[j, h + HIST, pl.ds(L, L)]
      acc[h, pl.ds(0, L)] = lo
      acc[h, pl.ds(L, L)] = hi
      return carry
    lax.fori_loop(0, HIST, hbody, 0)

  fire(0, 0)

  def gbody(i, carry):
    g = i * 2
    fire(1, g + 1)
    drain(0)
    accum(0)

    @pl.when(g + 2 < NSTEP)
    def _():
      fire(0, g + 2)
    drain(1)
    accum(1)
    return carry
  lax.fori_loop(0, NSTEP // 2, gbody, 0)

  plsc.subcore_barrier()
  pltpu.sync_copy(acc, shared.at[hidx_v], add=True)
  plsc.subcore_barrier()

  @pl.when(s == 0)
  def _():
    pltpu.sync_copy(shared, acc)
    scale = jnp.full((L,), 1.0 / BATCH, jnp.float32)

    def sbody(h, carry):
      acc[h, pl.ds(0, L)] = acc[h, pl.ds(0, L)] * scale
      acc[h, pl.ds(L, L)] = acc[h, pl.ds(L, L)] * scale
      return carry
    lax.fori_loop(0, HIST, sbody, 0)
    pltpu.sync_copy(acc.at[pl.ds(0, HIST)], out_hbm.at[c])


_sc_call = functools.partial(
    pl.kernel,
    out_type=jax.ShapeDtypeStruct((NC, HIST, EMBED_DIM), jnp.float32),
    mesh=plsc.VectorSubcoreMesh(core_axis_name="c", subcore_axis_name="s"),
    compiler_params=pltpu.CompilerParams(use_tc_tiling_on_sc=False),
    scratch_types=[
        pltpu.VMEM((2, NBLK, BLKI), jnp.int32),
        pltpu.VMEM((2, NBLK, BLKI, EMBED_DIM), jnp.float32),
        pltpu.VMEM((HPAD, EMBED_DIM), jnp.float32),
        pltpu.VMEM((HPAD,), jnp.int32),
        pltpu.VMEM_SHARED((HPAD, EMBED_DIM), jnp.float32),
        pltpu.SemaphoreType.DMA,
        pltpu.SemaphoreType.DMA,
    ],
)(_body)


@jax.jit
def kernel(input, table):
  idx = input.reshape(NW * NSTEP, NBLK, BLKI)
  hidx = jnp.arange(HPAD, dtype=jnp.int32)
  tab_flat = jax.lax.optimization_barrier(table.reshape(-1))
  tab_lin = tab_flat.reshape(VOCAB, EMBED_DIM)
  partial = _sc_call(idx, tab_lin, hidx)
  return partial.sum(axis=0)
